# Initial kernel scaffold; baseline (speedup 1.0000x reference)
import jax
import jax.numpy as jnp
from jax.experimental import pallas as pl


def kernel(x, masked_nodes, pos_edge_index, neg_edge_index, edge_index, W1, b1, W2, b2):
    raise NotImplementedError("write your pallas kernel here")


# speedup vs baseline: 50.0812x; 50.0812x over previous
import functools

import jax
import jax.numpy as jnp
from jax import lax
from jax.experimental import pallas as pl
from jax.experimental.pallas import tpu as pltpu
from jax.experimental.pallas import tpu_sc as plsc

NC, NS, L = 2, 16, 16
NW = NC * NS
CH = 80
NBUF = 5


def _mesh():
    return plsc.VectorSubcoreMesh(core_axis_name="c", subcore_axis_name="s")


_SC_PARAMS = pltpu.CompilerParams(use_tc_tiling_on_sc=False,
                                  needs_layout_passes=False)


def _make_deg(N, E):
    cpt = E // NW // CH

    @functools.partial(
        pl.kernel,
        out_type=jax.ShapeDtypeStruct((NC * N,), jnp.float32),
        mesh=_mesh(),
        compiler_params=_SC_PARAMS,
        scratch_types=[
            pltpu.VMEM((cpt, CH), jnp.int32),
            pltpu.VMEM((CH,), jnp.float32),
            pltpu.VMEM((N,), jnp.float32),
            pltpu.VMEM_SHARED((N,), jnp.float32),
            pltpu.SemaphoreType.DMA,
        ],
    )
    def deg_kernel(dst_hbm, zeros_hbm, out_hbm, dst_v, ones_v, dbuf, acc, dsem):
        c = lax.axis_index("c")
        s = lax.axis_index("s")
        pltpu.sync_copy(dst_hbm.at[wid := c * NS + s], dst_v)
        for i in range(CH // L):
            ones_v[pl.ds(i * L, L)] = jnp.ones((L,), jnp.float32)

        @pl.when(s == 0)
        def _():
            pltpu.sync_copy(zeros_hbm, dbuf)
            pltpu.sync_copy(dbuf, acc)

        plsc.subcore_barrier()

        def chunk(i, carry):
            pltpu.async_copy(ones_v, acc.at[dst_v.at[i]], dsem, add=True)
            return carry

        lax.fori_loop(0, cpt, chunk, 0)

        def drain(i, carry):
            pltpu.make_async_copy(ones_v, acc.at[dst_v.at[i]], dsem).wait()
            return carry

        lax.fori_loop(0, cpt, drain, 0)
        plsc.subcore_barrier()

        @pl.when(s == 0)
        def _():
            pltpu.sync_copy(acc, dbuf)
            pltpu.sync_copy(dbuf,
                            out_hbm.at[pl.ds(pl.multiple_of(c * N, 8), N)])

    return deg_kernel


def _make_agg(N, E, W):
    cpt = E // NW // CH
    base = (N // NS) // 8 * 8
    last = N - base * (NS - 1)

    @functools.partial(
        pl.kernel,
        out_type=jax.ShapeDtypeStruct((NC, N, W), jnp.float32),
        mesh=_mesh(),
        compiler_params=_SC_PARAMS,
        scratch_types=[
            pltpu.VMEM((cpt, CH), jnp.int32),
            pltpu.VMEM((cpt, CH), jnp.int32),
            pltpu.VMEM((NBUF, CH, W), jnp.float32),
            pltpu.VMEM((last, W), jnp.float32),
            pltpu.VMEM_SHARED((N, W), jnp.float32),
            pltpu.SemaphoreType.DMA((NBUF,)),
            pltpu.SemaphoreType.DMA((NBUF,)),
        ],
    )
    def agg_kernel(hp_hbm, src_hbm, dst_hbm, zeros_hbm, out_hbm,
                   src_v, dst_v, gbuf, bounce, acc, gsem, ssem):
        c = lax.axis_index("c")
        s = lax.axis_index("s")
        wid = c * NS + s
        pltpu.sync_copy(src_hbm.at[wid], src_v)
        pltpu.sync_copy(dst_hbm.at[wid], dst_v)

        @pl.when(s < NS - 1)
        def _():
            off = pl.multiple_of(s * base, 8)
            pltpu.sync_copy(zeros_hbm.at[pl.ds(off, base)],
                            bounce.at[pl.ds(0, base)])
            pltpu.sync_copy(bounce.at[pl.ds(0, base)],
                            acc.at[pl.ds(off, base)])

        @pl.when(s == NS - 1)
        def _():
            pltpu.sync_copy(zeros_hbm.at[pl.ds(base * (NS - 1), last)], bounce)
            pltpu.sync_copy(bounce, acc.at[pl.ds(base * (NS - 1), last)])

        plsc.subcore_barrier()

        for b in range(NBUF):
            pltpu.async_copy(hp_hbm.at[src_v.at[b]], gbuf.at[b], gsem.at[b])

        def round_(j, carry):
            for b in range(NBUF):
                i = j * NBUF + b
                pltpu.make_async_copy(hp_hbm.at[src_v.at[i]], gbuf.at[b],
                                      gsem.at[b]).wait()
                pltpu.async_copy(gbuf.at[b], acc.at[dst_v.at[i]], ssem.at[b],
                                 add=True)
                pltpu.make_async_copy(gbuf.at[b], acc.at[dst_v.at[i]],
                                      ssem.at[b]).wait()
                pltpu.async_copy(hp_hbm.at[src_v.at[i + NBUF]], gbuf.at[b],
                                 gsem.at[b])
            return carry

        lax.fori_loop(0, cpt // NBUF - 1, round_, 0)
        for b in range(NBUF):
            i = cpt - NBUF + b
            pltpu.make_async_copy(hp_hbm.at[src_v.at[i]], gbuf.at[b],
                                  gsem.at[b]).wait()
            pltpu.sync_copy(gbuf.at[b], acc.at[dst_v.at[i]], add=True)
        plsc.subcore_barrier()

        @pl.when(s < NS - 1)
        def _():
            off = pl.multiple_of(s * base, 8)
            pltpu.sync_copy(acc.at[pl.ds(off, base)],
                            bounce.at[pl.ds(0, base)])
            pltpu.sync_copy(bounce.at[pl.ds(0, base)],
                            out_hbm.at[c, pl.ds(off, base)])

        @pl.when(s == NS - 1)
        def _():
            pltpu.sync_copy(acc.at[pl.ds(base * (NS - 1), last)], bounce)
            pltpu.sync_copy(bounce, out_hbm.at[c, pl.ds(base * (NS - 1), last)])

    return agg_kernel


def _make_score(N, PE, C):
    ept = (2 * PE) // NW
    nb = ept // L
    hc = C // 2

    @functools.partial(
        pl.kernel,
        out_type=jax.ShapeDtypeStruct((2 * PE,), jnp.float32),
        mesh=_mesh(),
        compiler_params=_SC_PARAMS,
        scratch_types=[
            pltpu.VMEM((ept,), jnp.int32),
            pltpu.VMEM((ept,), jnp.int32),
            pltpu.VMEM((ept,), jnp.float32),
            pltpu.VMEM((hc, N), jnp.float32),
        ],
    )
    def score_kernel(h2t_hbm, pos_s_hbm, pos_d_hbm, neg_s_hbm, neg_d_hbm,
                     out_hbm, src_v, dst_v, acc_v, half_v):
        c = lax.axis_index("c")
        s = lax.axis_index("s")
        wid = c * NS + s

        @pl.when(wid < NS)
        def _():
            off = pl.multiple_of(wid * ept, 8)
            pltpu.sync_copy(pos_s_hbm.at[pl.ds(off, ept)], src_v)
            pltpu.sync_copy(pos_d_hbm.at[pl.ds(off, ept)], dst_v)

        @pl.when(wid >= NS)
        def _():
            off = pl.multiple_of((wid - NS) * ept, 8)
            pltpu.sync_copy(neg_s_hbm.at[pl.ds(off, ept)], src_v)
            pltpu.sync_copy(neg_d_hbm.at[pl.ds(off, ept)], dst_v)

        for half in range(2):
            pltpu.sync_copy(h2t_hbm.at[pl.ds(half * hc, hc)], half_v)

            def block(b, carry):
                si = src_v[pl.ds(b * L, L)]
                di = dst_v[pl.ds(b * L, L)]
                ss = jnp.zeros((L,), jnp.float32)
                for f in range(hc):
                    vi = plsc.load_gather(half_v.at[f], [di])
                    vj = plsc.load_gather(half_v.at[f], [si])
                    ss = ss + vi * vj
                if half == 0:
                    acc_v[pl.ds(b * L, L)] = ss
                else:
                    acc_v[pl.ds(b * L, L)] = acc_v[pl.ds(b * L, L)] + ss
                return carry

            lax.fori_loop(0, nb, block, 0)

        pltpu.sync_copy(acc_v, out_hbm.at[pl.ds(pl.multiple_of(wid * ept, 8),
                                                ept)])

    return score_kernel


def _mm1_body(x_ref, w_ref, degp_ref, hp_ref, dinv_ref):
    deg = degp_ref[0, :] + degp_ref[1, :] + 1.0
    dinv = lax.rsqrt(deg)
    h = jnp.dot(x_ref[...], w_ref[...], preferred_element_type=jnp.float32)
    hp_ref[...] = h * dinv[:, None]
    dinv_ref[...] = dinv[:, None]


def _mm2_body(acc_ref, hp_ref, dinv_ref, b1_ref, w2_ref, hp2_ref):
    dinv = dinv_ref[...]
    h1 = (acc_ref[0] + acc_ref[1] + hp_ref[...]) * dinv + b1_ref[...][None, :]
    h1 = jnp.maximum(h1, 0.0)
    h2 = jnp.dot(h1, w2_ref[...], preferred_element_type=jnp.float32)
    hp2_ref[...] = h2 * dinv


def _fin_body(acc_ref, hp2_ref, dinv_ref, b2_ref, h2_ref):
    dinv = dinv_ref[...]
    h2_ref[...] = ((acc_ref[0] + acc_ref[1] + hp2_ref[...]) * dinv
                   + b2_ref[...][None, :])


def kernel(x, masked_nodes, pos_edge_index, neg_edge_index, edge_index,
           W1, b1, W2, b2):
    N, F = x.shape
    H = W1.shape[1]
    C = W2.shape[1]
    E = edge_index.shape[1]
    PE = pos_edge_index.shape[1]
    assert E % (NW * CH) == 0 and N % NS == 0 and (2 * PE) % (NW * L) == 0
    assert (E // (NW * CH)) % NBUF == 0

    cpt = E // NW // CH
    src_r = edge_index[0].reshape(NW, cpt, CH)
    dst_r = edge_index[1].reshape(NW, cpt, CH)

    deg_p = _make_deg(N, E)(dst_r, jnp.zeros((N,), jnp.float32))
    deg_p = deg_p.reshape(NC, N)

    hp1, dinv = pl.pallas_call(
        _mm1_body,
        out_shape=[jax.ShapeDtypeStruct((N, H), jnp.float32),
                   jax.ShapeDtypeStruct((N, 1), jnp.float32)],
    )(x, W1, deg_p)

    acc1 = _make_agg(N, E, H)(hp1, src_r, dst_r, jnp.zeros((N, H), jnp.float32))

    hp2 = pl.pallas_call(
        _mm2_body,
        out_shape=jax.ShapeDtypeStruct((N, C), jnp.float32),
    )(acc1, hp1, dinv, b1, W2)

    acc2 = _make_agg(N, E, C)(hp2, src_r, dst_r, jnp.zeros((N, C), jnp.float32))

    h2 = pl.pallas_call(
        _fin_body,
        out_shape=jax.ShapeDtypeStruct((N, C), jnp.float32),
    )(acc2, hp2, dinv, b2)

    score = _make_score(N, PE, C)(
        h2.T, pos_edge_index[0], pos_edge_index[1],
        neg_edge_index[0], neg_edge_index[1])
    return (score, jnp.zeros((2, 2 * PE), jnp.float32))


# speedup vs baseline: 51.8493x; 1.0353x over previous
import functools

import jax
import jax.numpy as jnp
from jax import lax
from jax.experimental import pallas as pl
from jax.experimental.pallas import tpu as pltpu
from jax.experimental.pallas import tpu_sc as plsc

NC, NS, L = 2, 16, 16
NW = NC * NS
CH = 80
NBUF = 5


def _mesh():
    return plsc.VectorSubcoreMesh(core_axis_name="c", subcore_axis_name="s")


_SC_PARAMS = pltpu.CompilerParams(use_tc_tiling_on_sc=False,
                                  needs_layout_passes=False)


def _make_deg(N, E):
    ept = E // NW
    cpt = ept // CH

    @functools.partial(
        pl.kernel,
        out_type=jax.ShapeDtypeStruct((NC * N,), jnp.float32),
        mesh=_mesh(),
        compiler_params=_SC_PARAMS,
        scratch_types=[
            pltpu.VMEM((ept,), jnp.int32),
            pltpu.VMEM((CH,), jnp.float32),
            pltpu.VMEM((N,), jnp.float32),
            pltpu.VMEM_SHARED((N,), jnp.float32),
            pltpu.SemaphoreType.DMA,
        ],
    )
    def deg_kernel(dst_hbm, zeros_hbm, out_hbm, dst_v, ones_v, dbuf, acc, dsem):
        c = lax.axis_index("c")
        s = lax.axis_index("s")
        wid = c * NS + s
        pltpu.sync_copy(dst_hbm.at[pl.ds(pl.multiple_of(wid * ept, 8), ept)],
                        dst_v)
        for i in range(CH // L):
            ones_v[pl.ds(i * L, L)] = jnp.ones((L,), jnp.float32)

        @pl.when(s == 0)
        def _():
            pltpu.sync_copy(zeros_hbm, dbuf)
            pltpu.sync_copy(dbuf, acc)

        plsc.subcore_barrier()

        def chunk(i, carry):
            pltpu.async_copy(ones_v, acc.at[dst_v.at[pl.ds(i * CH, CH)]],
                             dsem, add=True)
            return carry

        lax.fori_loop(0, cpt, chunk, 0)

        def drain(i, carry):
            pltpu.make_async_copy(ones_v, acc.at[dst_v.at[pl.ds(i * CH, CH)]],
                                  dsem).wait()
            return carry

        lax.fori_loop(0, cpt, drain, 0)
        plsc.subcore_barrier()

        @pl.when(s == 0)
        def _():
            pltpu.sync_copy(acc, dbuf)
            pltpu.sync_copy(dbuf,
                            out_hbm.at[pl.ds(pl.multiple_of(c * N, 8), N)])

    return deg_kernel


def _make_agg(N, E, W):
    ept = E // NW
    cpt = ept // CH
    base = (N // NS) // 8 * 8
    last = N - base * (NS - 1)

    @functools.partial(
        pl.kernel,
        out_type=jax.ShapeDtypeStruct((NC, N, W), jnp.float32),
        mesh=_mesh(),
        compiler_params=_SC_PARAMS,
        scratch_types=[
            pltpu.VMEM((ept,), jnp.int32),
            pltpu.VMEM((ept,), jnp.int32),
            pltpu.VMEM((NBUF, CH, W), jnp.float32),
            pltpu.VMEM((last, W), jnp.float32),
            pltpu.VMEM_SHARED((N, W), jnp.float32),
            pltpu.SemaphoreType.DMA((NBUF,)),
            pltpu.SemaphoreType.DMA((NBUF,)),
        ],
    )
    def agg_kernel(hp_hbm, src_hbm, dst_hbm, zeros_hbm, out_hbm,
                   src_v, dst_v, gbuf, bounce, acc, gsem, ssem):
        c = lax.axis_index("c")
        s = lax.axis_index("s")
        wid = c * NS + s
        eoff = pl.multiple_of(wid * ept, 8)
        pltpu.sync_copy(src_hbm.at[pl.ds(eoff, ept)], src_v)

        for b in range(NBUF):
            pltpu.async_copy(hp_hbm.at[src_v.at[pl.ds(b * CH, CH)]],
                             gbuf.at[b], gsem.at[b])

        pltpu.sync_copy(dst_hbm.at[pl.ds(eoff, ept)], dst_v)

        @pl.when(s < NS - 1)
        def _():
            off = pl.multiple_of(s * base, 8)
            pltpu.sync_copy(zeros_hbm.at[pl.ds(off, base)],
                            bounce.at[pl.ds(0, base)])
            pltpu.sync_copy(bounce.at[pl.ds(0, base)],
                            acc.at[pl.ds(off, base)])

        @pl.when(s == NS - 1)
        def _():
            pltpu.sync_copy(zeros_hbm.at[pl.ds(base * (NS - 1), last)], bounce)
            pltpu.sync_copy(bounce, acc.at[pl.ds(base * (NS - 1), last)])

        plsc.subcore_barrier()

        def round_(j, carry):
            for b in range(NBUF):
                i = j * NBUF + b
                ic = pl.ds(i * CH, CH)
                nc_ = pl.ds((i + NBUF) * CH, CH)
                pltpu.make_async_copy(hp_hbm.at[src_v.at[ic]], gbuf.at[b],
                                      gsem.at[b]).wait()
                pltpu.async_copy(gbuf.at[b], acc.at[dst_v.at[ic]], ssem.at[b],
                                 add=True)
                pltpu.make_async_copy(gbuf.at[b], acc.at[dst_v.at[ic]],
                                      ssem.at[b]).wait()
                pltpu.async_copy(hp_hbm.at[src_v.at[nc_]], gbuf.at[b],
                                 gsem.at[b])
            return carry

        lax.fori_loop(0, cpt // NBUF - 1, round_, 0)
        for b in range(NBUF):
            ic = pl.ds((cpt - NBUF + b) * CH, CH)
            pltpu.make_async_copy(hp_hbm.at[src_v.at[ic]], gbuf.at[b],
                                  gsem.at[b]).wait()
            pltpu.sync_copy(gbuf.at[b], acc.at[dst_v.at[ic]], add=True)
        plsc.subcore_barrier()

        @pl.when(s < NS - 1)
        def _():
            off = pl.multiple_of(s * base, 8)
            pltpu.sync_copy(acc.at[pl.ds(off, base)],
                            bounce.at[pl.ds(0, base)])
            pltpu.sync_copy(bounce.at[pl.ds(0, base)],
                            out_hbm.at[c, pl.ds(off, base)])

        @pl.when(s == NS - 1)
        def _():
            pltpu.sync_copy(acc.at[pl.ds(base * (NS - 1), last)], bounce)
            pltpu.sync_copy(bounce, out_hbm.at[c, pl.ds(base * (NS - 1), last)])

    return agg_kernel


def _make_score(N, PE, C):
    ept = (2 * PE) // NW
    nb = ept // L
    npass = 4
    hc = C // npass
    base = (N // NS) // 8 * 8
    last = N - base * (NS - 1)

    @functools.partial(
        pl.kernel,
        out_type=jax.ShapeDtypeStruct((2 * PE,), jnp.float32),
        mesh=_mesh(),
        compiler_params=_SC_PARAMS,
        scratch_types=[
            pltpu.VMEM((ept,), jnp.int32),
            pltpu.VMEM((ept,), jnp.int32),
            pltpu.VMEM((ept,), jnp.float32),
            pltpu.VMEM((hc, N), jnp.float32),
            pltpu.VMEM((last, C), jnp.float32),
            pltpu.VMEM((last, C), jnp.float32),
            pltpu.VMEM((last, C), jnp.float32),
            pltpu.VMEM((last,), jnp.float32),
            pltpu.VMEM((C,), jnp.float32),
            pltpu.VMEM((C, last), jnp.float32),
            pltpu.VMEM_SHARED((C, N), jnp.float32),
        ],
    )
    def score_kernel(acc2_hbm, hp2_hbm, dinv_hbm, b2_hbm,
                     pos_s_hbm, pos_d_hbm, neg_s_hbm, neg_d_hbm,
                     out_hbm, src_v, dst_v, acc_v, half_v,
                     a0_v, a1_v, hp_v, dv_v, b2_v, lt_v, h2t):
        c = lax.axis_index("c")
        s = lax.axis_index("s")
        wid = c * NS + s

        @pl.when(wid < NS)
        def _():
            off = pl.multiple_of(wid * ept, 8)
            pltpu.sync_copy(pos_s_hbm.at[pl.ds(off, ept)], src_v)
            pltpu.sync_copy(pos_d_hbm.at[pl.ds(off, ept)], dst_v)

        @pl.when(wid >= NS)
        def _():
            off = pl.multiple_of((wid - NS) * ept, 8)
            pltpu.sync_copy(neg_s_hbm.at[pl.ds(off, ept)], src_v)
            pltpu.sync_copy(neg_d_hbm.at[pl.ds(off, ept)], dst_v)

        pltpu.sync_copy(b2_hbm, b2_v)

        @pl.when(s < NS - 1)
        def _():
            off = pl.multiple_of(s * base, 8)
            pltpu.sync_copy(acc2_hbm.at[0, pl.ds(off, base)],
                            a0_v.at[pl.ds(0, base)])
            pltpu.sync_copy(acc2_hbm.at[1, pl.ds(off, base)],
                            a1_v.at[pl.ds(0, base)])
            pltpu.sync_copy(hp2_hbm.at[pl.ds(off, base)],
                            hp_v.at[pl.ds(0, base)])
            pltpu.sync_copy(dinv_hbm.at[pl.ds(off, base)],
                            dv_v.at[pl.ds(0, base)])

        @pl.when(s == NS - 1)
        def _():
            off = base * (NS - 1)
            pltpu.sync_copy(acc2_hbm.at[0, pl.ds(off, last)], a0_v)
            pltpu.sync_copy(acc2_hbm.at[1, pl.ds(off, last)], a1_v)
            pltpu.sync_copy(hp2_hbm.at[pl.ds(off, last)], hp_v)
            pltpu.sync_copy(dinv_hbm.at[pl.ds(off, last)], dv_v)

        ngrp = lax.select(s == NS - 1, last // L, base // L)
        b2row = b2_v[...]
        fidx = lax.iota(jnp.int32, L)
        zil = jnp.zeros((L,), jnp.int32)

        def node16(g, carry):
            dvec = dv_v[pl.ds(g * L, L)]
            for k in range(L):
                j = g * L + k
                row = (a0_v[j] + a1_v[j] + hp_v[j]) * dvec[k] + b2row
                plsc.store_scatter(lt_v, [fidx, zil + j], row)
            return carry

        lax.fori_loop(0, ngrp, node16, 0)

        @pl.when(s < NS - 1)
        def _():
            off = pl.multiple_of(s * base, 8)
            pltpu.sync_copy(lt_v.at[:, pl.ds(0, base)],
                            h2t.at[:, pl.ds(off, base)])

        @pl.when(s == NS - 1)
        def _():
            pltpu.sync_copy(lt_v, h2t.at[:, pl.ds(base * (NS - 1), last)])

        plsc.subcore_barrier()

        for half in range(npass):
            pltpu.sync_copy(h2t.at[pl.ds(half * hc, hc)], half_v)

            def block(b, carry):
                si = src_v[pl.ds(b * L, L)]
                di = dst_v[pl.ds(b * L, L)]
                ss = jnp.zeros((L,), jnp.float32)
                for f in range(hc):
                    vi = plsc.load_gather(half_v.at[f], [di])
                    vj = plsc.load_gather(half_v.at[f], [si])
                    ss = ss + vi * vj
                if half == 0:
                    acc_v[pl.ds(b * L, L)] = ss
                else:
                    acc_v[pl.ds(b * L, L)] = acc_v[pl.ds(b * L, L)] + ss
                return carry

            lax.fori_loop(0, nb, block, 0)

        pltpu.sync_copy(acc_v, out_hbm.at[pl.ds(pl.multiple_of(wid * ept, 8),
                                                ept)])

    return score_kernel


def _mma_body(x_ref, w_ref, h_ref):
    h_ref[...] = jnp.dot(x_ref[...], w_ref[...],
                         preferred_element_type=jnp.float32)


def _mmb_body(h_ref, degp_ref, hp_ref, dinv2_ref, dinv1_ref):
    deg = degp_ref[0, :] + degp_ref[1, :] + 1.0
    dinv = lax.rsqrt(deg)
    hp_ref[...] = h_ref[...] * dinv[:, None]
    dinv2_ref[...] = dinv[:, None]
    dinv1_ref[...] = dinv


def _mm2_body(acc_ref, hp_ref, dinv_ref, b1_ref, w2_ref, hp2_ref):
    dinv = dinv_ref[...]
    h1 = (acc_ref[0] + acc_ref[1] + hp_ref[...]) * dinv + b1_ref[...][None, :]
    h1 = jnp.maximum(h1, 0.0)
    h2 = jnp.dot(h1, w2_ref[...], preferred_element_type=jnp.float32)
    hp2_ref[...] = h2 * dinv


def kernel(x, masked_nodes, pos_edge_index, neg_edge_index, edge_index,
           W1, b1, W2, b2):
    N, F = x.shape
    H = W1.shape[1]
    C = W2.shape[1]
    E = edge_index.shape[1]
    PE = pos_edge_index.shape[1]
    assert E % (NW * CH) == 0 and N % NS == 0 and (2 * PE) % (NW * L) == 0
    assert (E // (NW * CH)) % NBUF == 0 and C == L

    src1d = edge_index[0]
    dst1d = edge_index[1]

    h1raw = pl.pallas_call(
        _mma_body,
        out_shape=jax.ShapeDtypeStruct((N, H), jnp.float32),
    )(x, W1)

    deg_p = _make_deg(N, E)(dst1d, jnp.zeros((N,), jnp.float32))

    hp1, dinv2, dinv1 = pl.pallas_call(
        _mmb_body,
        out_shape=[jax.ShapeDtypeStruct((N, H), jnp.float32),
                   jax.ShapeDtypeStruct((N, 1), jnp.float32),
                   jax.ShapeDtypeStruct((N,), jnp.float32)],
    )(h1raw, deg_p.reshape(NC, N))

    acc1 = _make_agg(N, E, H)(hp1, src1d, dst1d, jnp.zeros((N, H), jnp.float32))

    hp2 = pl.pallas_call(
        _mm2_body,
        out_shape=jax.ShapeDtypeStruct((N, C), jnp.float32),
    )(acc1, hp1, dinv2, b1, W2)

    acc2 = _make_agg(N, E, C)(hp2, src1d, dst1d, jnp.zeros((N, C), jnp.float32))

    score = _make_score(N, PE, C)(
        acc2, hp2, dinv1, b2,
        pos_edge_index[0], pos_edge_index[1],
        neg_edge_index[0], neg_edge_index[1])
    return (score, jnp.zeros((2, 2 * PE), jnp.float32))


# speedup vs baseline: 54.2975x; 1.0472x over previous
import functools

import jax
import jax.numpy as jnp
from jax import lax
from jax.experimental import pallas as pl
from jax.experimental.pallas import tpu as pltpu
from jax.experimental.pallas import tpu_sc as plsc

NC, NS, L = 2, 16, 16
NW = NC * NS
CH = 80
NBUF = 5


def _mesh():
    return plsc.VectorSubcoreMesh(core_axis_name="c", subcore_axis_name="s")


_SC_PARAMS = pltpu.CompilerParams(use_tc_tiling_on_sc=False,
                                  needs_layout_passes=False)


def _make_deg(N, E):
    ept = E // NW
    cpt = ept // CH

    @functools.partial(
        pl.kernel,
        out_type=jax.ShapeDtypeStruct((NC * N,), jnp.float32),
        mesh=_mesh(),
        compiler_params=_SC_PARAMS,
        scratch_types=[
            pltpu.VMEM((ept,), jnp.int32),
            pltpu.VMEM((CH,), jnp.float32),
            pltpu.VMEM((N,), jnp.float32),
            pltpu.VMEM_SHARED((N,), jnp.float32),
            pltpu.SemaphoreType.DMA,
        ],
    )
    def deg_kernel(edge_hbm, zeros_hbm, out_hbm, dst_v, ones_v, dbuf, acc, dsem):
        c = lax.axis_index("c")
        s = lax.axis_index("s")
        wid = c * NS + s
        pltpu.sync_copy(edge_hbm.at[1, pl.ds(pl.multiple_of(wid * ept, 8), ept)],
                        dst_v)
        for i in range(CH // L):
            ones_v[pl.ds(i * L, L)] = jnp.ones((L,), jnp.float32)

        @pl.when(s == 0)
        def _():
            pltpu.sync_copy(zeros_hbm, dbuf)
            pltpu.sync_copy(dbuf, acc)

        plsc.subcore_barrier()

        def chunk(i, carry):
            pltpu.async_copy(ones_v, acc.at[dst_v.at[pl.ds(i * CH, CH)]],
                             dsem, add=True)
            return carry

        lax.fori_loop(0, cpt, chunk, 0)

        def drain(i, carry):
            pltpu.make_async_copy(ones_v, acc.at[dst_v.at[pl.ds(i * CH, CH)]],
                                  dsem).wait()
            return carry

        lax.fori_loop(0, cpt, drain, 0)
        plsc.subcore_barrier()

        @pl.when(s == 0)
        def _():
            pltpu.sync_copy(acc, dbuf)
            pltpu.sync_copy(dbuf,
                            out_hbm.at[pl.ds(pl.multiple_of(c * N, 8), N)])

    return deg_kernel


def _make_agg(N, E, W):
    ept = E // NW
    cpt = ept // CH
    base = (N // NS) // 8 * 8
    last = N - base * (NS - 1)

    @functools.partial(
        pl.kernel,
        out_type=jax.ShapeDtypeStruct((NC, N, W), jnp.float32),
        mesh=_mesh(),
        compiler_params=_SC_PARAMS,
        scratch_types=[
            pltpu.VMEM((ept,), jnp.int32),
            pltpu.VMEM((ept,), jnp.int32),
            pltpu.VMEM((NBUF, CH, W), jnp.float32),
            pltpu.VMEM((last, W), jnp.float32),
            pltpu.VMEM_SHARED((N, W), jnp.float32),
            pltpu.SemaphoreType.DMA((NBUF,)),
            pltpu.SemaphoreType.DMA((NBUF,)),
        ],
    )
    def agg_kernel(hp_hbm, edge_hbm, zeros_hbm, out_hbm,
                   src_v, dst_v, gbuf, bounce, acc, gsem, ssem):
        c = lax.axis_index("c")
        s = lax.axis_index("s")
        wid = c * NS + s
        eoff = pl.multiple_of(wid * ept, 8)
        pltpu.sync_copy(edge_hbm.at[0, pl.ds(eoff, ept)], src_v)

        for b in range(NBUF):
            pltpu.async_copy(hp_hbm.at[src_v.at[pl.ds(b * CH, CH)]],
                             gbuf.at[b], gsem.at[b])

        pltpu.sync_copy(edge_hbm.at[1, pl.ds(eoff, ept)], dst_v)

        @pl.when(s < NS - 1)
        def _():
            off = pl.multiple_of(s * base, 8)
            pltpu.sync_copy(zeros_hbm.at[pl.ds(off, base)],
                            bounce.at[pl.ds(0, base)])
            pltpu.sync_copy(bounce.at[pl.ds(0, base)],
                            acc.at[pl.ds(off, base)])

        @pl.when(s == NS - 1)
        def _():
            pltpu.sync_copy(zeros_hbm.at[pl.ds(base * (NS - 1), last)], bounce)
            pltpu.sync_copy(bounce, acc.at[pl.ds(base * (NS - 1), last)])

        plsc.subcore_barrier()

        def round_(j, carry):
            for b in range(NBUF):
                i = j * NBUF + b
                ic = pl.ds(i * CH, CH)
                nc_ = pl.ds((i + NBUF) * CH, CH)
                pltpu.make_async_copy(hp_hbm.at[src_v.at[ic]], gbuf.at[b],
                                      gsem.at[b]).wait()
                pltpu.async_copy(gbuf.at[b], acc.at[dst_v.at[ic]], ssem.at[b],
                                 add=True)
                pltpu.make_async_copy(gbuf.at[b], acc.at[dst_v.at[ic]],
                                      ssem.at[b]).wait()
                pltpu.async_copy(hp_hbm.at[src_v.at[nc_]], gbuf.at[b],
                                 gsem.at[b])
            return carry

        lax.fori_loop(0, cpt // NBUF - 1, round_, 0)
        for b in range(NBUF):
            ic = pl.ds((cpt - NBUF + b) * CH, CH)
            pltpu.make_async_copy(hp_hbm.at[src_v.at[ic]], gbuf.at[b],
                                  gsem.at[b]).wait()
            pltpu.sync_copy(gbuf.at[b], acc.at[dst_v.at[ic]], add=True)
        plsc.subcore_barrier()

        @pl.when(s < NS - 1)
        def _():
            off = pl.multiple_of(s * base, 8)
            pltpu.sync_copy(acc.at[pl.ds(off, base)],
                            bounce.at[pl.ds(0, base)])
            pltpu.sync_copy(bounce.at[pl.ds(0, base)],
                            out_hbm.at[c, pl.ds(off, base)])

        @pl.when(s == NS - 1)
        def _():
            pltpu.sync_copy(acc.at[pl.ds(base * (NS - 1), last)], bounce)
            pltpu.sync_copy(bounce, out_hbm.at[c, pl.ds(base * (NS - 1), last)])

    return agg_kernel


def _make_score(N, PE, C):
    ept = (2 * PE) // NW
    nb = ept // L
    npass = 4
    hc = C // npass
    base = (N // NS) // 8 * 8
    last = N - base * (NS - 1)

    @functools.partial(
        pl.kernel,
        out_type=jax.ShapeDtypeStruct((2 * PE,), jnp.float32),
        mesh=_mesh(),
        compiler_params=_SC_PARAMS,
        scratch_types=[
            pltpu.VMEM((ept,), jnp.int32),
            pltpu.VMEM((ept,), jnp.int32),
            pltpu.VMEM((ept,), jnp.float32),
            pltpu.VMEM((hc, N), jnp.float32),
            pltpu.VMEM((last, C), jnp.float32),
            pltpu.VMEM((last, C), jnp.float32),
            pltpu.VMEM((last, C), jnp.float32),
            pltpu.VMEM((last,), jnp.float32),
            pltpu.VMEM((C,), jnp.float32),
            pltpu.VMEM((C, last), jnp.float32),
            pltpu.VMEM_SHARED((C, N), jnp.float32),
        ],
    )
    def score_kernel(acc2_hbm, hp2_hbm, dinv_hbm, b2_hbm,
                     pos_hbm, neg_hbm,
                     out_hbm, src_v, dst_v, acc_v, half_v,
                     a0_v, a1_v, hp_v, dv_v, b2_v, lt_v, h2t):
        c = lax.axis_index("c")
        s = lax.axis_index("s")
        wid = c * NS + s

        @pl.when(wid < NS)
        def _():
            off = pl.multiple_of(wid * ept, 8)
            pltpu.sync_copy(pos_hbm.at[0, pl.ds(off, ept)], src_v)
            pltpu.sync_copy(pos_hbm.at[1, pl.ds(off, ept)], dst_v)

        @pl.when(wid >= NS)
        def _():
            off = pl.multiple_of((wid - NS) * ept, 8)
            pltpu.sync_copy(neg_hbm.at[0, pl.ds(off, ept)], src_v)
            pltpu.sync_copy(neg_hbm.at[1, pl.ds(off, ept)], dst_v)

        pltpu.sync_copy(b2_hbm, b2_v)

        @pl.when(s < NS - 1)
        def _():
            off = pl.multiple_of(s * base, 8)
            pltpu.sync_copy(acc2_hbm.at[0, pl.ds(off, base)],
                            a0_v.at[pl.ds(0, base)])
            pltpu.sync_copy(acc2_hbm.at[1, pl.ds(off, base)],
                            a1_v.at[pl.ds(0, base)])
            pltpu.sync_copy(hp2_hbm.at[pl.ds(off, base)],
                            hp_v.at[pl.ds(0, base)])
            pltpu.sync_copy(dinv_hbm.at[pl.ds(off, base)],
                            dv_v.at[pl.ds(0, base)])

        @pl.when(s == NS - 1)
        def _():
            off = base * (NS - 1)
            pltpu.sync_copy(acc2_hbm.at[0, pl.ds(off, last)], a0_v)
            pltpu.sync_copy(acc2_hbm.at[1, pl.ds(off, last)], a1_v)
            pltpu.sync_copy(hp2_hbm.at[pl.ds(off, last)], hp_v)
            pltpu.sync_copy(dinv_hbm.at[pl.ds(off, last)], dv_v)

        ngrp = lax.select(s == NS - 1, last // L, base // L)
        b2row = b2_v[...]
        fidx = lax.iota(jnp.int32, L)
        zil = jnp.zeros((L,), jnp.int32)

        def node16(g, carry):
            dvec = dv_v[pl.ds(g * L, L)]
            for k in range(L):
                j = g * L + k
                row = (a0_v[j] + a1_v[j] + hp_v[j]) * dvec[k] + b2row
                plsc.store_scatter(lt_v, [fidx, zil + j], row)
            return carry

        lax.fori_loop(0, ngrp, node16, 0)

        @pl.when(s < NS - 1)
        def _():
            off = pl.multiple_of(s * base, 8)
            pltpu.sync_copy(lt_v.at[:, pl.ds(0, base)],
                            h2t.at[:, pl.ds(off, base)])

        @pl.when(s == NS - 1)
        def _():
            pltpu.sync_copy(lt_v, h2t.at[:, pl.ds(base * (NS - 1), last)])

        plsc.subcore_barrier()

        for half in range(npass):
            pltpu.sync_copy(h2t.at[pl.ds(half * hc, hc)], half_v)

            def one_block(b):
                si = src_v[pl.ds(b * L, L)]
                di = dst_v[pl.ds(b * L, L)]
                ss = jnp.zeros((L,), jnp.float32)
                for f in range(hc):
                    vi = plsc.load_gather(half_v.at[f], [di])
                    vj = plsc.load_gather(half_v.at[f], [si])
                    ss = ss + vi * vj
                if half == 0:
                    acc_v[pl.ds(b * L, L)] = ss
                else:
                    acc_v[pl.ds(b * L, L)] = acc_v[pl.ds(b * L, L)] + ss

            def block(b2_, carry):
                for k in range(2):
                    one_block(b2_ * 2 + k)
                return carry

            lax.fori_loop(0, nb // 2, block, 0)
            for b in range(nb // 2 * 2, nb):
                one_block(b)

        pltpu.sync_copy(acc_v, out_hbm.at[pl.ds(pl.multiple_of(wid * ept, 8),
                                                ept)])

    return score_kernel


def _mma_body(x_ref, w_ref, h_ref):
    h_ref[...] = jnp.dot(x_ref[...], w_ref[...],
                         preferred_element_type=jnp.float32)


def _mmb_body(h_ref, degp_ref, hp_ref, dinv2_ref, dinv1_ref):
    deg = degp_ref[0, :] + degp_ref[1, :] + 1.0
    dinv = lax.rsqrt(deg)
    hp_ref[...] = h_ref[...] * dinv[:, None]
    dinv2_ref[...] = dinv[:, None]
    dinv1_ref[...] = dinv


def _mm2_body(acc_ref, hp_ref, dinv_ref, b1_ref, w2_ref, hp2_ref):
    dinv = dinv_ref[...]
    h1 = (acc_ref[0] + acc_ref[1] + hp_ref[...]) * dinv + b1_ref[...][None, :]
    h1 = jnp.maximum(h1, 0.0)
    h2 = jnp.dot(h1, w2_ref[...], preferred_element_type=jnp.float32)
    hp2_ref[...] = h2 * dinv


def kernel(x, masked_nodes, pos_edge_index, neg_edge_index, edge_index,
           W1, b1, W2, b2):
    N, F = x.shape
    H = W1.shape[1]
    C = W2.shape[1]
    E = edge_index.shape[1]
    PE = pos_edge_index.shape[1]
    assert E % (NW * CH) == 0 and N % NS == 0 and (2 * PE) % (NW * L) == 0
    assert (E // (NW * CH)) % NBUF == 0 and C == L

    h1raw = pl.pallas_call(
        _mma_body,
        out_shape=jax.ShapeDtypeStruct((N, H), jnp.float32),
    )(x, W1)

    deg_p = _make_deg(N, E)(edge_index, jnp.zeros((N,), jnp.float32))

    hp1, dinv2, dinv1 = pl.pallas_call(
        _mmb_body,
        out_shape=[jax.ShapeDtypeStruct((N, H), jnp.float32),
                   jax.ShapeDtypeStruct((N, 1), jnp.float32),
                   jax.ShapeDtypeStruct((N,), jnp.float32)],
    )(h1raw, deg_p.reshape(NC, N))

    acc1 = _make_agg(N, E, H)(hp1, edge_index, jnp.zeros((N, H), jnp.float32))

    hp2 = pl.pallas_call(
        _mm2_body,
        out_shape=jax.ShapeDtypeStruct((N, C), jnp.float32),
    )(acc1, hp1, dinv2, b1, W2)

    acc2 = _make_agg(N, E, C)(hp2, edge_index, jnp.zeros((N, C), jnp.float32))

    score = _make_score(N, PE, C)(
        acc2, hp2, dinv1, b2, pos_edge_index, neg_edge_index)
    return (score, jnp.zeros((2, 2 * PE), jnp.float32))


# speedup vs baseline: 57.1660x; 1.0528x over previous
import functools

import jax
import jax.numpy as jnp
from jax import lax
from jax.experimental import pallas as pl
from jax.experimental.pallas import tpu as pltpu
from jax.experimental.pallas import tpu_sc as plsc

NC, NS, L = 2, 16, 16
NW = NC * NS
CH = 80
NBUF = 5


def _mesh():
    return plsc.VectorSubcoreMesh(core_axis_name="c", subcore_axis_name="s")


_SC_PARAMS = pltpu.CompilerParams(use_tc_tiling_on_sc=False,
                                  needs_layout_passes=False)


def _make_deg(N, E):
    ept = E // NW
    cpt = ept // CH

    @functools.partial(
        pl.kernel,
        out_type=jax.ShapeDtypeStruct((NC * N,), jnp.float32),
        mesh=_mesh(),
        compiler_params=_SC_PARAMS,
        scratch_types=[
            pltpu.VMEM((ept,), jnp.int32),
            pltpu.VMEM((CH,), jnp.float32),
            pltpu.VMEM((N,), jnp.float32),
            pltpu.VMEM_SHARED((N,), jnp.float32),
            pltpu.SemaphoreType.DMA,
        ],
    )
    def deg_kernel(edge_hbm, zeros_hbm, out_hbm, dst_v, ones_v, dbuf, acc, dsem):
        c = lax.axis_index("c")
        s = lax.axis_index("s")
        wid = c * NS + s
        pltpu.sync_copy(edge_hbm.at[1, pl.ds(pl.multiple_of(wid * ept, 8), ept)],
                        dst_v)
        for i in range(CH // L):
            ones_v[pl.ds(i * L, L)] = jnp.ones((L,), jnp.float32)

        @pl.when(s == 0)
        def _():
            pltpu.sync_copy(zeros_hbm, dbuf)
            pltpu.sync_copy(dbuf, acc)

        plsc.subcore_barrier()

        def chunk(i, carry):
            pltpu.async_copy(ones_v, acc.at[dst_v.at[pl.ds(i * CH, CH)]],
                             dsem, add=True)
            return carry

        lax.fori_loop(0, cpt, chunk, 0)

        def drain(i, carry):
            pltpu.make_async_copy(ones_v, acc.at[dst_v.at[pl.ds(i * CH, CH)]],
                                  dsem).wait()
            return carry

        lax.fori_loop(0, cpt, drain, 0)
        plsc.subcore_barrier()

        @pl.when(s == 0)
        def _():
            pltpu.sync_copy(acc, dbuf)
            pltpu.sync_copy(dbuf,
                            out_hbm.at[pl.ds(pl.multiple_of(c * N, 8), N)])

    return deg_kernel


def _make_agg(N, E, W):
    ept = E // NW
    cpt = ept // CH
    base = (N // NS) // 8 * 8
    last = N - base * (NS - 1)

    @functools.partial(
        pl.kernel,
        out_type=jax.ShapeDtypeStruct((NC, N, W), jnp.float32),
        mesh=_mesh(),
        compiler_params=_SC_PARAMS,
        scratch_types=[
            pltpu.VMEM((ept,), jnp.int32),
            pltpu.VMEM((ept,), jnp.int32),
            pltpu.VMEM((NBUF, CH, W), jnp.float32),
            pltpu.VMEM((last, W), jnp.float32),
            pltpu.VMEM_SHARED((N, W), jnp.float32),
            pltpu.SemaphoreType.DMA((NBUF,)),
            pltpu.SemaphoreType.DMA((NBUF,)),
        ],
    )
    def agg_kernel(hp_hbm, edge_hbm, zeros_hbm, out_hbm,
                   src_v, dst_v, gbuf, bounce, acc, gsem, ssem):
        c = lax.axis_index("c")
        s = lax.axis_index("s")
        wid = c * NS + s
        eoff = pl.multiple_of(wid * ept, 8)
        pltpu.sync_copy(edge_hbm.at[0, pl.ds(eoff, ept)], src_v)

        for b in range(NBUF):
            pltpu.async_copy(hp_hbm.at[src_v.at[pl.ds(b * CH, CH)]],
                             gbuf.at[b], gsem.at[b])

        pltpu.sync_copy(edge_hbm.at[1, pl.ds(eoff, ept)], dst_v)

        @pl.when(s < NS - 1)
        def _():
            off = pl.multiple_of(s * base, 8)
            pltpu.sync_copy(zeros_hbm.at[pl.ds(off, base)],
                            bounce.at[pl.ds(0, base)])
            pltpu.sync_copy(bounce.at[pl.ds(0, base)],
                            acc.at[pl.ds(off, base)])

        @pl.when(s == NS - 1)
        def _():
            pltpu.sync_copy(zeros_hbm.at[pl.ds(base * (NS - 1), last)], bounce)
            pltpu.sync_copy(bounce, acc.at[pl.ds(base * (NS - 1), last)])

        plsc.subcore_barrier()

        def round_(j, carry):
            for b in range(NBUF):
                i = j * NBUF + b
                ic = pl.ds(i * CH, CH)
                nc_ = pl.ds((i + NBUF) * CH, CH)
                pltpu.make_async_copy(hp_hbm.at[src_v.at[ic]], gbuf.at[b],
                                      gsem.at[b]).wait()
                pltpu.async_copy(gbuf.at[b], acc.at[dst_v.at[ic]], ssem.at[b],
                                 add=True)
                pltpu.make_async_copy(gbuf.at[b], acc.at[dst_v.at[ic]],
                                      ssem.at[b]).wait()
                pltpu.async_copy(hp_hbm.at[src_v.at[nc_]], gbuf.at[b],
                                 gsem.at[b])
            return carry

        lax.fori_loop(0, cpt // NBUF - 1, round_, 0)
        for b in range(NBUF):
            ic = pl.ds((cpt - NBUF + b) * CH, CH)
            pltpu.make_async_copy(hp_hbm.at[src_v.at[ic]], gbuf.at[b],
                                  gsem.at[b]).wait()
            pltpu.sync_copy(gbuf.at[b], acc.at[dst_v.at[ic]], add=True)
        plsc.subcore_barrier()

        @pl.when(s < NS - 1)
        def _():
            off = pl.multiple_of(s * base, 8)
            pltpu.sync_copy(acc.at[pl.ds(off, base)],
                            bounce.at[pl.ds(0, base)])
            pltpu.sync_copy(bounce.at[pl.ds(0, base)],
                            out_hbm.at[c, pl.ds(off, base)])

        @pl.when(s == NS - 1)
        def _():
            pltpu.sync_copy(acc.at[pl.ds(base * (NS - 1), last)], bounce)
            pltpu.sync_copy(bounce, out_hbm.at[c, pl.ds(base * (NS - 1), last)])

    return agg_kernel


def _make_score(N, PE, C):
    ept = (2 * PE) // NW
    nb = ept // L
    npass = 2
    hc = C // npass
    base = (N // NS) // 8 * 8
    last = N - base * (NS - 1)

    @functools.partial(
        pl.kernel,
        out_type=jax.ShapeDtypeStruct((2 * PE,), jnp.float32),
        mesh=_mesh(),
        compiler_params=_SC_PARAMS,
        scratch_types=[
            pltpu.VMEM((ept,), jnp.int32),
            pltpu.VMEM((ept,), jnp.int32),
            pltpu.VMEM((ept,), jnp.float32),
            pltpu.VMEM_SHARED((C, N), jnp.float32),
        ],
    )
    def score_kernel(acc2_hbm, hp2_hbm, dinv_hbm, b2_hbm,
                     pos_hbm, neg_hbm,
                     out_hbm, src_v, dst_v, acc_v, h2t):
        c = lax.axis_index("c")
        s = lax.axis_index("s")
        wid = c * NS + s

        @pl.when(wid < NS)
        def _():
            off = pl.multiple_of(wid * ept, 8)
            pltpu.sync_copy(pos_hbm.at[0, pl.ds(off, ept)], src_v)
            pltpu.sync_copy(pos_hbm.at[1, pl.ds(off, ept)], dst_v)

        @pl.when(wid >= NS)
        def _():
            off = pl.multiple_of((wid - NS) * ept, 8)
            pltpu.sync_copy(neg_hbm.at[0, pl.ds(off, ept)], src_v)
            pltpu.sync_copy(neg_hbm.at[1, pl.ds(off, ept)], dst_v)

        def build(a0_v, a1_v, hp_v, dv_v, b2_v, lt_v):
            pltpu.sync_copy(b2_hbm, b2_v)

            @pl.when(s < NS - 1)
            def _():
                off = pl.multiple_of(s * base, 8)
                pltpu.sync_copy(acc2_hbm.at[0, pl.ds(off, base)],
                                a0_v.at[pl.ds(0, base)])
                pltpu.sync_copy(acc2_hbm.at[1, pl.ds(off, base)],
                                a1_v.at[pl.ds(0, base)])
                pltpu.sync_copy(hp2_hbm.at[pl.ds(off, base)],
                                hp_v.at[pl.ds(0, base)])
                pltpu.sync_copy(dinv_hbm.at[pl.ds(off, base)],
                                dv_v.at[pl.ds(0, base)])

            @pl.when(s == NS - 1)
            def _():
                off = base * (NS - 1)
                pltpu.sync_copy(acc2_hbm.at[0, pl.ds(off, last)], a0_v)
                pltpu.sync_copy(acc2_hbm.at[1, pl.ds(off, last)], a1_v)
                pltpu.sync_copy(hp2_hbm.at[pl.ds(off, last)], hp_v)
                pltpu.sync_copy(dinv_hbm.at[pl.ds(off, last)], dv_v)

            ngrp = lax.select(s == NS - 1, last // L, base // L)
            b2row = b2_v[...]
            fidx = lax.iota(jnp.int32, L)
            zil = jnp.zeros((L,), jnp.int32)

            def node16(g, carry):
                dvec = dv_v[pl.ds(g * L, L)]
                for k in range(L):
                    j = g * L + k
                    row = (a0_v[j] + a1_v[j] + hp_v[j]) * dvec[k] + b2row
                    plsc.store_scatter(lt_v, [fidx, zil + j], row)
                return carry

            lax.fori_loop(0, ngrp, node16, 0)

            @pl.when(s < NS - 1)
            def _():
                off = pl.multiple_of(s * base, 8)
                pltpu.sync_copy(lt_v.at[:, pl.ds(0, base)],
                                h2t.at[:, pl.ds(off, base)])

            @pl.when(s == NS - 1)
            def _():
                pltpu.sync_copy(lt_v, h2t.at[:, pl.ds(base * (NS - 1), last)])

        pl.run_scoped(build,
                      pltpu.VMEM((last, C), jnp.float32),
                      pltpu.VMEM((last, C), jnp.float32),
                      pltpu.VMEM((last, C), jnp.float32),
                      pltpu.VMEM((last,), jnp.float32),
                      pltpu.VMEM((C,), jnp.float32),
                      pltpu.VMEM((C, last), jnp.float32))

        plsc.subcore_barrier()

        def gather_phase(half_v):
            for half in range(npass):
                pltpu.sync_copy(h2t.at[pl.ds(half * hc, hc)], half_v)

                def one_block(b):
                    si = src_v[pl.ds(b * L, L)]
                    di = dst_v[pl.ds(b * L, L)]
                    ss = jnp.zeros((L,), jnp.float32)
                    for f in range(hc):
                        vi = plsc.load_gather(half_v.at[f], [di])
                        vj = plsc.load_gather(half_v.at[f], [si])
                        ss = ss + vi * vj
                    if half == 0:
                        acc_v[pl.ds(b * L, L)] = ss
                    else:
                        acc_v[pl.ds(b * L, L)] = acc_v[pl.ds(b * L, L)] + ss

                def block(b2_, carry):
                    for k in range(2):
                        one_block(b2_ * 2 + k)
                    return carry

                lax.fori_loop(0, nb // 2, block, 0)
                for b in range(nb // 2 * 2, nb):
                    one_block(b)

        pl.run_scoped(gather_phase, pltpu.VMEM((hc, N), jnp.float32))

        pltpu.sync_copy(acc_v, out_hbm.at[pl.ds(pl.multiple_of(wid * ept, 8),
                                                ept)])

    return score_kernel


def _mma_body(x_ref, w_ref, h_ref):
    h_ref[...] = jnp.dot(x_ref[...], w_ref[...],
                         preferred_element_type=jnp.float32)


def _mmb_body(h_ref, degp_ref, hp_ref, dinv2_ref, dinv1_ref):
    deg = degp_ref[0, :] + degp_ref[1, :] + 1.0
    dinv = lax.rsqrt(deg)
    hp_ref[...] = h_ref[...] * dinv[:, None]
    dinv2_ref[...] = dinv[:, None]
    dinv1_ref[...] = dinv


def _mm2_body(acc_ref, hp_ref, dinv_ref, b1_ref, w2_ref, hp2_ref):
    dinv = dinv_ref[...]
    h1 = (acc_ref[...] + hp_ref[...]) * dinv + b1_ref[...][None, :]
    h1 = jnp.maximum(h1, 0.0)
    h2 = jnp.dot(h1, w2_ref[...], preferred_element_type=jnp.float32)
    hp2_ref[...] = h2 * dinv


def kernel(x, masked_nodes, pos_edge_index, neg_edge_index, edge_index,
           W1, b1, W2, b2):
    N, F = x.shape
    H = W1.shape[1]
    C = W2.shape[1]
    E = edge_index.shape[1]
    PE = pos_edge_index.shape[1]
    assert E % (NW * CH) == 0 and N % NS == 0 and (2 * PE) % (NW * L) == 0
    assert (E // (NW * CH)) % NBUF == 0 and C == L

    h1raw = pl.pallas_call(
        _mma_body,
        out_shape=jax.ShapeDtypeStruct((N, H), jnp.float32),
    )(x, W1)

    deg_p = _make_deg(N, E)(edge_index, jnp.zeros((N,), jnp.float32))

    hp1, dinv2, dinv1 = pl.pallas_call(
        _mmb_body,
        out_shape=[jax.ShapeDtypeStruct((N, H), jnp.float32),
                   jax.ShapeDtypeStruct((N, 1), jnp.float32),
                   jax.ShapeDtypeStruct((N,), jnp.float32)],
    )(h1raw, deg_p.reshape(NC, N))

    acc1 = _make_agg(N, E, H)(hp1, edge_index, jnp.zeros((N, H), jnp.float32))

    hp2 = pl.pallas_call(
        _mm2_body,
        out_shape=jax.ShapeDtypeStruct((N, C), jnp.float32),
    )(acc1[0] + acc1[1], hp1, dinv2, b1, W2)

    acc2 = _make_agg(N, E, C)(hp2, edge_index, jnp.zeros((N, C), jnp.float32))

    score = _make_score(N, PE, C)(
        acc2, hp2, dinv1, b2, pos_edge_index, neg_edge_index)
    return (score, jnp.zeros((2, 2 * PE), jnp.float32))


# speedup vs baseline: 58.1429x; 1.0171x over previous
import functools

import jax
import jax.numpy as jnp
from jax import lax
from jax.experimental import pallas as pl
from jax.experimental.pallas import tpu as pltpu
from jax.experimental.pallas import tpu_sc as plsc

NC, NS, L = 2, 16, 16
NW = NC * NS
CH = 80
NBUF = 5


def _mesh():
    return plsc.VectorSubcoreMesh(core_axis_name="c", subcore_axis_name="s")


_SC_PARAMS = pltpu.CompilerParams(use_tc_tiling_on_sc=False,
                                  needs_layout_passes=False)


def _make_deg(N, E):
    ept = E // NW
    cpt = ept // CH

    @functools.partial(
        pl.kernel,
        out_type=jax.ShapeDtypeStruct((NC * N,), jnp.float32),
        mesh=_mesh(),
        compiler_params=_SC_PARAMS,
        scratch_types=[
            pltpu.VMEM((ept,), jnp.int32),
            pltpu.VMEM((CH,), jnp.float32),
            pltpu.VMEM((N,), jnp.float32),
            pltpu.VMEM_SHARED((N,), jnp.float32),
            pltpu.SemaphoreType.DMA,
        ],
    )
    def deg_kernel(edge_hbm, out_hbm, dst_v, ones_v, dbuf, acc, dsem):
        c = lax.axis_index("c")
        s = lax.axis_index("s")
        wid = c * NS + s
        pltpu.sync_copy(edge_hbm.at[1, pl.ds(pl.multiple_of(wid * ept, 8), ept)],
                        dst_v)
        for i in range(CH // L):
            ones_v[pl.ds(i * L, L)] = jnp.ones((L,), jnp.float32)

        @pl.when(s == 0)
        def _():
            def zrow(i, carry):
                dbuf[pl.ds(i * L, L)] = jnp.zeros((L,), jnp.float32)
                return carry

            lax.fori_loop(0, N // L, zrow, 0)
            pltpu.sync_copy(dbuf, acc)

        plsc.subcore_barrier()

        def chunk(i, carry):
            pltpu.async_copy(ones_v, acc.at[dst_v.at[pl.ds(i * CH, CH)]],
                             dsem, add=True)
            return carry

        lax.fori_loop(0, cpt, chunk, 0)

        def drain(i, carry):
            pltpu.make_async_copy(ones_v, acc.at[dst_v.at[pl.ds(i * CH, CH)]],
                                  dsem).wait()
            return carry

        lax.fori_loop(0, cpt, drain, 0)
        plsc.subcore_barrier()

        @pl.when(s == 0)
        def _():
            pltpu.sync_copy(acc, dbuf)
            pltpu.sync_copy(dbuf,
                            out_hbm.at[pl.ds(pl.multiple_of(c * N, 8), N)])

    return deg_kernel


def _make_agg(N, E, W):
    ept = E // NW
    cpt = ept // CH
    base = (N // NS) // 8 * 8
    last = N - base * (NS - 1)

    @functools.partial(
        pl.kernel,
        out_type=jax.ShapeDtypeStruct((NC, N, W), jnp.float32),
        mesh=_mesh(),
        compiler_params=_SC_PARAMS,
        scratch_types=[
            pltpu.VMEM((ept,), jnp.int32),
            pltpu.VMEM((ept,), jnp.int32),
            pltpu.VMEM((NBUF, CH, W), jnp.float32),
            pltpu.VMEM((last, W), jnp.float32),
            pltpu.VMEM_SHARED((N, W), jnp.float32),
            pltpu.SemaphoreType.DMA((NBUF,)),
            pltpu.SemaphoreType.DMA((NBUF,)),
        ],
    )
    def agg_kernel(hp_hbm, edge_hbm, out_hbm,
                   src_v, dst_v, gbuf, bounce, acc, gsem, ssem):
        c = lax.axis_index("c")
        s = lax.axis_index("s")
        wid = c * NS + s
        eoff = pl.multiple_of(wid * ept, 8)
        pltpu.sync_copy(edge_hbm.at[0, pl.ds(eoff, ept)], src_v)

        for b in range(NBUF):
            pltpu.async_copy(hp_hbm.at[src_v.at[pl.ds(b * CH, CH)]],
                             gbuf.at[b], gsem.at[b])

        pltpu.sync_copy(edge_hbm.at[1, pl.ds(eoff, ept)], dst_v)

        def zrow(i, carry):
            for k in range(W // L):
                bounce[i, pl.ds(k * L, L)] = jnp.zeros((L,), jnp.float32)
            return carry

        lax.fori_loop(0, last, zrow, 0)

        @pl.when(s < NS - 1)
        def _():
            off = pl.multiple_of(s * base, 8)
            pltpu.sync_copy(bounce.at[pl.ds(0, base)],
                            acc.at[pl.ds(off, base)])

        @pl.when(s == NS - 1)
        def _():
            pltpu.sync_copy(bounce, acc.at[pl.ds(base * (NS - 1), last)])

        plsc.subcore_barrier()

        def round_(j, carry):
            for b in range(NBUF):
                i = j * NBUF + b
                ic = pl.ds(i * CH, CH)
                nc_ = pl.ds((i + NBUF) * CH, CH)
                pltpu.make_async_copy(hp_hbm.at[src_v.at[ic]], gbuf.at[b],
                                      gsem.at[b]).wait()
                pltpu.async_copy(gbuf.at[b], acc.at[dst_v.at[ic]], ssem.at[b],
                                 add=True)
                pltpu.make_async_copy(gbuf.at[b], acc.at[dst_v.at[ic]],
                                      ssem.at[b]).wait()
                pltpu.async_copy(hp_hbm.at[src_v.at[nc_]], gbuf.at[b],
                                 gsem.at[b])
            return carry

        lax.fori_loop(0, cpt // NBUF - 1, round_, 0)
        for b in range(NBUF):
            ic = pl.ds((cpt - NBUF + b) * CH, CH)
            pltpu.make_async_copy(hp_hbm.at[src_v.at[ic]], gbuf.at[b],
                                  gsem.at[b]).wait()
            pltpu.sync_copy(gbuf.at[b], acc.at[dst_v.at[ic]], add=True)
        plsc.subcore_barrier()

        @pl.when(s < NS - 1)
        def _():
            off = pl.multiple_of(s * base, 8)
            pltpu.sync_copy(acc.at[pl.ds(off, base)],
                            bounce.at[pl.ds(0, base)])
            pltpu.sync_copy(bounce.at[pl.ds(0, base)],
                            out_hbm.at[c, pl.ds(off, base)])

        @pl.when(s == NS - 1)
        def _():
            pltpu.sync_copy(acc.at[pl.ds(base * (NS - 1), last)], bounce)
            pltpu.sync_copy(bounce, out_hbm.at[c, pl.ds(base * (NS - 1), last)])

    return agg_kernel


def _make_score(N, PE, C):
    ept = (2 * PE) // NW
    nb = ept // L
    npass = 2
    hc = C // npass
    base = (N // NS) // 8 * 8
    last = N - base * (NS - 1)

    @functools.partial(
        pl.kernel,
        out_type=jax.ShapeDtypeStruct((2 * PE,), jnp.float32),
        mesh=_mesh(),
        compiler_params=_SC_PARAMS,
        scratch_types=[
            pltpu.VMEM((ept,), jnp.int32),
            pltpu.VMEM((ept,), jnp.int32),
            pltpu.VMEM((ept,), jnp.float32),
            pltpu.VMEM_SHARED((C, N), jnp.float32),
        ],
    )
    def score_kernel(acc2_hbm, hp2_hbm, dinv_hbm, b2_hbm,
                     pos_hbm, neg_hbm,
                     out_hbm, src_v, dst_v, acc_v, h2t):
        c = lax.axis_index("c")
        s = lax.axis_index("s")
        wid = c * NS + s

        @pl.when(wid < NS)
        def _():
            off = pl.multiple_of(wid * ept, 8)
            pltpu.sync_copy(pos_hbm.at[0, pl.ds(off, ept)], src_v)
            pltpu.sync_copy(pos_hbm.at[1, pl.ds(off, ept)], dst_v)

        @pl.when(wid >= NS)
        def _():
            off = pl.multiple_of((wid - NS) * ept, 8)
            pltpu.sync_copy(neg_hbm.at[0, pl.ds(off, ept)], src_v)
            pltpu.sync_copy(neg_hbm.at[1, pl.ds(off, ept)], dst_v)

        def build(a0_v, a1_v, hp_v, dv_v, b2_v, lt_v):
            pltpu.sync_copy(b2_hbm, b2_v)

            @pl.when(s < NS - 1)
            def _():
                off = pl.multiple_of(s * base, 8)
                pltpu.sync_copy(acc2_hbm.at[0, pl.ds(off, base)],
                                a0_v.at[pl.ds(0, base)])
                pltpu.sync_copy(acc2_hbm.at[1, pl.ds(off, base)],
                                a1_v.at[pl.ds(0, base)])
                pltpu.sync_copy(hp2_hbm.at[pl.ds(off, base)],
                                hp_v.at[pl.ds(0, base)])
                pltpu.sync_copy(dinv_hbm.at[pl.ds(off, base)],
                                dv_v.at[pl.ds(0, base)])

            @pl.when(s == NS - 1)
            def _():
                off = base * (NS - 1)
                pltpu.sync_copy(acc2_hbm.at[0, pl.ds(off, last)], a0_v)
                pltpu.sync_copy(acc2_hbm.at[1, pl.ds(off, last)], a1_v)
                pltpu.sync_copy(hp2_hbm.at[pl.ds(off, last)], hp_v)
                pltpu.sync_copy(dinv_hbm.at[pl.ds(off, last)], dv_v)

            ngrp = lax.select(s == NS - 1, last // L, base // L)
            b2row = b2_v[...]
            fidx = lax.iota(jnp.int32, L)
            zil = jnp.zeros((L,), jnp.int32)

            def node16(g, carry):
                dvec = dv_v[pl.ds(g * L, L)]
                for k in range(L):
                    j = g * L + k
                    row = (a0_v[j] + a1_v[j] + hp_v[j]) * dvec[k] + b2row
                    plsc.store_scatter(lt_v, [fidx, zil + j], row)
                return carry

            lax.fori_loop(0, ngrp, node16, 0)

            @pl.when(s < NS - 1)
            def _():
                off = pl.multiple_of(s * base, 8)
                pltpu.sync_copy(lt_v.at[:, pl.ds(0, base)],
                                h2t.at[:, pl.ds(off, base)])

            @pl.when(s == NS - 1)
            def _():
                pltpu.sync_copy(lt_v, h2t.at[:, pl.ds(base * (NS - 1), last)])

        pl.run_scoped(build,
                      pltpu.VMEM((last, C), jnp.float32),
                      pltpu.VMEM((last, C), jnp.float32),
                      pltpu.VMEM((last, C), jnp.float32),
                      pltpu.VMEM((last,), jnp.float32),
                      pltpu.VMEM((C,), jnp.float32),
                      pltpu.VMEM((C, last), jnp.float32))

        plsc.subcore_barrier()

        def gather_phase(half_v):
            for half in range(npass):
                pltpu.sync_copy(h2t.at[pl.ds(half * hc, hc)], half_v)

                def one_block(b):
                    si = src_v[pl.ds(b * L, L)]
                    di = dst_v[pl.ds(b * L, L)]
                    ss = jnp.zeros((L,), jnp.float32)
                    for f in range(hc):
                        vi = plsc.load_gather(half_v.at[f], [di])
                        vj = plsc.load_gather(half_v.at[f], [si])
                        ss = ss + vi * vj
                    if half == 0:
                        acc_v[pl.ds(b * L, L)] = ss
                    else:
                        acc_v[pl.ds(b * L, L)] = acc_v[pl.ds(b * L, L)] + ss

                def block(b2_, carry):
                    for k in range(2):
                        one_block(b2_ * 2 + k)
                    return carry

                lax.fori_loop(0, nb // 2, block, 0)
                for b in range(nb // 2 * 2, nb):
                    one_block(b)

        pl.run_scoped(gather_phase, pltpu.VMEM((hc, N), jnp.float32))

        pltpu.sync_copy(acc_v, out_hbm.at[pl.ds(pl.multiple_of(wid * ept, 8),
                                                ept)])

    return score_kernel


def _mma_body(x_ref, w_ref, h_ref):
    h_ref[...] = jnp.dot(x_ref[...], w_ref[...],
                         preferred_element_type=jnp.float32)


def _mmb_body(h_ref, degp_ref, hp_ref, dinv2_ref, dinv1_ref):
    deg = degp_ref[0, :] + degp_ref[1, :] + 1.0
    dinv = lax.rsqrt(deg)
    hp_ref[...] = h_ref[...] * dinv[:, None]
    dinv2_ref[...] = dinv[:, None]
    dinv1_ref[...] = dinv


def _mm2_body(acc_ref, hp_ref, dinv_ref, b1_ref, w2_ref, hp2_ref):
    dinv = dinv_ref[...]
    h1 = (acc_ref[0] + acc_ref[1] + hp_ref[...]) * dinv + b1_ref[...][None, :]
    h1 = jnp.maximum(h1, 0.0)
    h2 = jnp.dot(h1, w2_ref[...], preferred_element_type=jnp.float32)
    hp2_ref[...] = h2 * dinv


def kernel(x, masked_nodes, pos_edge_index, neg_edge_index, edge_index,
           W1, b1, W2, b2):
    N, F = x.shape
    H = W1.shape[1]
    C = W2.shape[1]
    E = edge_index.shape[1]
    PE = pos_edge_index.shape[1]
    assert E % (NW * CH) == 0 and N % NS == 0 and (2 * PE) % (NW * L) == 0
    assert (E // (NW * CH)) % NBUF == 0 and C == L

    h1raw = pl.pallas_call(
        _mma_body,
        out_shape=jax.ShapeDtypeStruct((N, H), jnp.float32),
    )(x, W1)

    deg_p = _make_deg(N, E)(edge_index)

    hp1, dinv2, dinv1 = pl.pallas_call(
        _mmb_body,
        out_shape=[jax.ShapeDtypeStruct((N, H), jnp.float32),
                   jax.ShapeDtypeStruct((N, 1), jnp.float32),
                   jax.ShapeDtypeStruct((N,), jnp.float32)],
    )(h1raw, deg_p.reshape(NC, N))

    acc1 = _make_agg(N, E, H)(hp1, edge_index)

    hp2 = pl.pallas_call(
        _mm2_body,
        out_shape=jax.ShapeDtypeStruct((N, C), jnp.float32),
    )(acc1, hp1, dinv2, b1, W2)

    acc2 = _make_agg(N, E, C)(hp2, edge_index)

    score = _make_score(N, PE, C)(
        acc2, hp2, dinv1, b2, pos_edge_index, neg_edge_index)
    return (score, jnp.zeros((2, 2 * PE), jnp.float32))


# speedup vs baseline: 60.1625x; 1.0347x over previous
import functools

import jax
import jax.numpy as jnp
from jax import lax
from jax.experimental import pallas as pl
from jax.experimental.pallas import tpu as pltpu
from jax.experimental.pallas import tpu_sc as plsc

NC, NS, L = 2, 16, 16
NW = NC * NS
CH = 80
NBUF = 5


def _mesh():
    return plsc.VectorSubcoreMesh(core_axis_name="c", subcore_axis_name="s")


_SC_PARAMS = pltpu.CompilerParams(use_tc_tiling_on_sc=False,
                                  needs_layout_passes=False)


def _make_deg(N, E):
    ept = E // NW
    cpt = ept // CH

    @functools.partial(
        pl.kernel,
        out_type=jax.ShapeDtypeStruct((NC * N,), jnp.float32),
        mesh=_mesh(),
        compiler_params=_SC_PARAMS,
        scratch_types=[
            pltpu.VMEM((ept,), jnp.int32),
            pltpu.VMEM((CH,), jnp.float32),
            pltpu.VMEM((N,), jnp.float32),
            pltpu.VMEM_SHARED((N,), jnp.float32),
            pltpu.SemaphoreType.DMA,
        ],
    )
    def deg_kernel(edge_hbm, out_hbm, dst_v, ones_v, dbuf, acc, dsem):
        c = lax.axis_index("c")
        s = lax.axis_index("s")
        wid = c * NS + s
        pltpu.sync_copy(edge_hbm.at[1, pl.ds(pl.multiple_of(wid * ept, 8), ept)],
                        dst_v)
        for i in range(CH // L):
            ones_v[pl.ds(i * L, L)] = jnp.ones((L,), jnp.float32)

        @pl.when(s == 0)
        def _():
            def zrow(i, carry):
                dbuf[pl.ds(i * L, L)] = jnp.zeros((L,), jnp.float32)
                return carry

            lax.fori_loop(0, N // L, zrow, 0)
            pltpu.sync_copy(dbuf, acc)

        plsc.subcore_barrier()

        def chunk(i, carry):
            pltpu.async_copy(ones_v, acc.at[dst_v.at[pl.ds(i * CH, CH)]],
                             dsem, add=True)
            return carry

        lax.fori_loop(0, cpt, chunk, 0)

        def drain(i, carry):
            pltpu.make_async_copy(ones_v, acc.at[dst_v.at[pl.ds(i * CH, CH)]],
                                  dsem).wait()
            return carry

        lax.fori_loop(0, cpt, drain, 0)
        plsc.subcore_barrier()

        @pl.when(s == 0)
        def _():
            pltpu.sync_copy(acc, dbuf)
            pltpu.sync_copy(dbuf,
                            out_hbm.at[pl.ds(pl.multiple_of(c * N, 8), N)])

    return deg_kernel


def _make_agg(N, E, W):
    ept = E // NW
    cpt = ept // CH
    base = (N // NS) // 8 * 8
    last = N - base * (NS - 1)

    @functools.partial(
        pl.kernel,
        out_type=jax.ShapeDtypeStruct((NC, N, W), jnp.float32),
        mesh=_mesh(),
        compiler_params=_SC_PARAMS,
        scratch_types=[
            pltpu.VMEM((ept,), jnp.int32),
            pltpu.VMEM((ept,), jnp.int32),
            pltpu.VMEM((NBUF, CH, W), jnp.float32),
            pltpu.VMEM((last, W), jnp.float32),
            pltpu.VMEM_SHARED((N, W), jnp.float32),
            pltpu.SemaphoreType.DMA((NBUF,)),
            pltpu.SemaphoreType.DMA((NBUF,)),
        ],
    )
    def agg_kernel(hp_hbm, edge_hbm, out_hbm,
                   src_v, dst_v, gbuf, bounce, acc, gsem, ssem):
        c = lax.axis_index("c")
        s = lax.axis_index("s")
        wid = c * NS + s
        eoff = pl.multiple_of(wid * ept, 8)
        pltpu.sync_copy(edge_hbm.at[0, pl.ds(eoff, ept)], src_v)

        for b in range(NBUF):
            pltpu.async_copy(hp_hbm.at[src_v.at[pl.ds(b * CH, CH)]],
                             gbuf.at[b], gsem.at[b])

        pltpu.sync_copy(edge_hbm.at[1, pl.ds(eoff, ept)], dst_v)

        def zrow(i, carry):
            for k in range(W // L):
                bounce[i, pl.ds(k * L, L)] = jnp.zeros((L,), jnp.float32)
            return carry

        lax.fori_loop(0, last, zrow, 0)

        @pl.when(s < NS - 1)
        def _():
            off = pl.multiple_of(s * base, 8)
            pltpu.sync_copy(bounce.at[pl.ds(0, base)],
                            acc.at[pl.ds(off, base)])

        @pl.when(s == NS - 1)
        def _():
            pltpu.sync_copy(bounce, acc.at[pl.ds(base * (NS - 1), last)])

        plsc.subcore_barrier()

        def round_(j, carry):
            for b in range(NBUF):
                i = j * NBUF + b
                ic = pl.ds(i * CH, CH)
                nc_ = pl.ds((i + NBUF) * CH, CH)
                pltpu.make_async_copy(hp_hbm.at[src_v.at[ic]], gbuf.at[b],
                                      gsem.at[b]).wait()
                pltpu.async_copy(gbuf.at[b], acc.at[dst_v.at[ic]], ssem.at[b],
                                 add=True)
                pltpu.make_async_copy(gbuf.at[b], acc.at[dst_v.at[ic]],
                                      ssem.at[b]).wait()
                pltpu.async_copy(hp_hbm.at[src_v.at[nc_]], gbuf.at[b],
                                 gsem.at[b])
            return carry

        lax.fori_loop(0, cpt // NBUF - 1, round_, 0)
        for b in range(NBUF):
            ic = pl.ds((cpt - NBUF + b) * CH, CH)
            pltpu.make_async_copy(hp_hbm.at[src_v.at[ic]], gbuf.at[b],
                                  gsem.at[b]).wait()
            pltpu.sync_copy(gbuf.at[b], acc.at[dst_v.at[ic]], add=True)
        plsc.subcore_barrier()

        @pl.when(s < NS - 1)
        def _():
            off = pl.multiple_of(s * base, 8)
            pltpu.sync_copy(acc.at[pl.ds(off, base)],
                            bounce.at[pl.ds(0, base)])
            pltpu.sync_copy(bounce.at[pl.ds(0, base)],
                            out_hbm.at[c, pl.ds(off, base)])

        @pl.when(s == NS - 1)
        def _():
            pltpu.sync_copy(acc.at[pl.ds(base * (NS - 1), last)], bounce)
            pltpu.sync_copy(bounce, out_hbm.at[c, pl.ds(base * (NS - 1), last)])

    return agg_kernel


def _make_score(N, PE, C):
    ept = (2 * PE) // NW
    nb = ept // L
    npass = 2
    hc = C // npass
    base = (N // NS) // 8 * 8
    last = N - base * (NS - 1)

    @functools.partial(
        pl.kernel,
        out_type=jax.ShapeDtypeStruct((2 * PE,), jnp.float32),
        mesh=_mesh(),
        compiler_params=_SC_PARAMS,
        scratch_types=[
            pltpu.VMEM((ept,), jnp.int32),
            pltpu.VMEM((ept,), jnp.int32),
            pltpu.VMEM((ept,), jnp.float32),
            pltpu.VMEM_SHARED((C, N), jnp.float32),
        ],
    )
    def score_kernel(acc2_hbm, hp2_hbm, dinv_hbm, b2_hbm,
                     pos_hbm, neg_hbm,
                     out_hbm, src_v, dst_v, acc_v, h2t):
        c = lax.axis_index("c")
        s = lax.axis_index("s")
        wid = c * NS + s

        @pl.when(wid < NS)
        def _():
            off = pl.multiple_of(wid * ept, 8)
            pltpu.sync_copy(pos_hbm.at[0, pl.ds(off, ept)], src_v)
            pltpu.sync_copy(pos_hbm.at[1, pl.ds(off, ept)], dst_v)

        @pl.when(wid >= NS)
        def _():
            off = pl.multiple_of((wid - NS) * ept, 8)
            pltpu.sync_copy(neg_hbm.at[0, pl.ds(off, ept)], src_v)
            pltpu.sync_copy(neg_hbm.at[1, pl.ds(off, ept)], dst_v)

        def build(a0_v, a1_v, hp_v, dv_v, b2_v, lt_v):
            pltpu.sync_copy(b2_hbm, b2_v)

            @pl.when(s < NS - 1)
            def _():
                off = pl.multiple_of(s * base, 8)
                pltpu.sync_copy(acc2_hbm.at[0, pl.ds(off, base)],
                                a0_v.at[pl.ds(0, base)])
                pltpu.sync_copy(acc2_hbm.at[1, pl.ds(off, base)],
                                a1_v.at[pl.ds(0, base)])
                pltpu.sync_copy(hp2_hbm.at[pl.ds(off, base)],
                                hp_v.at[pl.ds(0, base)])
                pltpu.sync_copy(dinv_hbm.at[pl.ds(off, base)],
                                dv_v.at[pl.ds(0, base)])

            @pl.when(s == NS - 1)
            def _():
                off = base * (NS - 1)
                pltpu.sync_copy(acc2_hbm.at[0, pl.ds(off, last)], a0_v)
                pltpu.sync_copy(acc2_hbm.at[1, pl.ds(off, last)], a1_v)
                pltpu.sync_copy(hp2_hbm.at[pl.ds(off, last)], hp_v)
                pltpu.sync_copy(dinv_hbm.at[pl.ds(off, last)], dv_v)

            ngrp = lax.select(s == NS - 1, last // L, base // L)
            b2row = b2_v[...]
            fidx = lax.iota(jnp.int32, L)
            zil = jnp.zeros((L,), jnp.int32)

            def node16(g, carry):
                dvec = dv_v[pl.ds(g * L, L)]
                for k in range(L):
                    j = g * L + k
                    row = (a0_v[j] + a1_v[j] + hp_v[j]) * dvec[k] + b2row
                    plsc.store_scatter(lt_v, [fidx, zil + j], row)
                return carry

            lax.fori_loop(0, ngrp, node16, 0)

            @pl.when(s < NS - 1)
            def _():
                off = pl.multiple_of(s * base, 8)
                pltpu.sync_copy(lt_v.at[:, pl.ds(0, base)],
                                h2t.at[:, pl.ds(off, base)])

            @pl.when(s == NS - 1)
            def _():
                pltpu.sync_copy(lt_v, h2t.at[:, pl.ds(base * (NS - 1), last)])

        pl.run_scoped(build,
                      pltpu.VMEM((last, C), jnp.float32),
                      pltpu.VMEM((last, C), jnp.float32),
                      pltpu.VMEM((last, C), jnp.float32),
                      pltpu.VMEM((last,), jnp.float32),
                      pltpu.VMEM((C,), jnp.float32),
                      pltpu.VMEM((C, last), jnp.float32))

        plsc.subcore_barrier()

        def gather_phase(half_v):
            for half in range(npass):
                pltpu.sync_copy(h2t.at[pl.ds(half * hc, hc)], half_v)

                def one_block(b):
                    si = src_v[pl.ds(b * L, L)]
                    di = dst_v[pl.ds(b * L, L)]
                    ss = jnp.zeros((L,), jnp.float32)
                    for f in range(hc):
                        vi = plsc.load_gather(half_v.at[f], [di])
                        vj = plsc.load_gather(half_v.at[f], [si])
                        ss = ss + vi * vj
                    if half == 0:
                        acc_v[pl.ds(b * L, L)] = ss
                    else:
                        acc_v[pl.ds(b * L, L)] = acc_v[pl.ds(b * L, L)] + ss

                def block(b2_):
                    for k in range(2):
                        one_block(b2_ * 2 + k)

                plsc.parallel_loop(0, nb // 2, 1, unroll=2)(block)
                for b in range(nb // 2 * 2, nb):
                    one_block(b)

        pl.run_scoped(gather_phase, pltpu.VMEM((hc, N), jnp.float32))

        pltpu.sync_copy(acc_v, out_hbm.at[pl.ds(pl.multiple_of(wid * ept, 8),
                                                ept)])

    return score_kernel


def _mma_body(x_ref, w_ref, h_ref):
    h_ref[...] = jnp.dot(x_ref[...], w_ref[...],
                         preferred_element_type=jnp.float32)


def _mmb_body(h_ref, degp_ref, hp_ref, dinv2_ref, dinv1_ref):
    deg = degp_ref[0, :] + degp_ref[1, :] + 1.0
    dinv = lax.rsqrt(deg)
    hp_ref[...] = h_ref[...] * dinv[:, None]
    dinv2_ref[...] = dinv[:, None]
    dinv1_ref[...] = dinv


def _mm2_body(acc_ref, hp_ref, dinv_ref, b1_ref, w2_ref, hp2_ref):
    dinv = dinv_ref[...]
    h1 = (acc_ref[0] + acc_ref[1] + hp_ref[...]) * dinv + b1_ref[...][None, :]
    h1 = jnp.maximum(h1, 0.0)
    h2 = jnp.dot(h1, w2_ref[...], preferred_element_type=jnp.float32)
    hp2_ref[...] = h2 * dinv


def kernel(x, masked_nodes, pos_edge_index, neg_edge_index, edge_index,
           W1, b1, W2, b2):
    N, F = x.shape
    H = W1.shape[1]
    C = W2.shape[1]
    E = edge_index.shape[1]
    PE = pos_edge_index.shape[1]
    assert E % (NW * CH) == 0 and N % NS == 0 and (2 * PE) % (NW * L) == 0
    assert (E // (NW * CH)) % NBUF == 0 and C == L

    h1raw = pl.pallas_call(
        _mma_body,
        out_shape=jax.ShapeDtypeStruct((N, H), jnp.float32),
    )(x, W1)

    deg_p = _make_deg(N, E)(edge_index)

    hp1, dinv2, dinv1 = pl.pallas_call(
        _mmb_body,
        out_shape=[jax.ShapeDtypeStruct((N, H), jnp.float32),
                   jax.ShapeDtypeStruct((N, 1), jnp.float32),
                   jax.ShapeDtypeStruct((N,), jnp.float32)],
    )(h1raw, deg_p.reshape(NC, N))

    acc1 = _make_agg(N, E, H)(hp1, edge_index)

    hp2 = pl.pallas_call(
        _mm2_body,
        out_shape=jax.ShapeDtypeStruct((N, C), jnp.float32),
    )(acc1, hp1, dinv2, b1, W2)

    acc2 = _make_agg(N, E, C)(hp2, edge_index)

    score = _make_score(N, PE, C)(
        acc2, hp2, dinv1, b2, pos_edge_index, neg_edge_index)
    return (score, jnp.zeros((2, 2 * PE), jnp.float32))


# speedup vs baseline: 60.2375x; 1.0012x over previous
import functools

import jax
import jax.numpy as jnp
from jax import lax
from jax.experimental import pallas as pl
from jax.experimental.pallas import tpu as pltpu
from jax.experimental.pallas import tpu_sc as plsc

NC, NS, L = 2, 16, 16
NW = NC * NS
CH = 80
NBUF = 5


def _mesh():
    return plsc.VectorSubcoreMesh(core_axis_name="c", subcore_axis_name="s")


_SC_PARAMS = pltpu.CompilerParams(use_tc_tiling_on_sc=False,
                                  needs_layout_passes=False)


def _make_deg(N, E):
    ept = E // NW
    cpt = ept // CH

    @functools.partial(
        pl.kernel,
        out_type=jax.ShapeDtypeStruct((NC * N,), jnp.float32),
        mesh=_mesh(),
        compiler_params=_SC_PARAMS,
        scratch_types=[
            pltpu.VMEM((ept,), jnp.int32),
            pltpu.VMEM((CH,), jnp.float32),
            pltpu.VMEM((N,), jnp.float32),
            pltpu.VMEM_SHARED((N,), jnp.float32),
            pltpu.SemaphoreType.DMA,
        ],
    )
    def deg_kernel(edge_hbm, out_hbm, dst_v, ones_v, dbuf, acc, dsem):
        c = lax.axis_index("c")
        s = lax.axis_index("s")
        wid = c * NS + s
        pltpu.sync_copy(edge_hbm.at[1, pl.ds(pl.multiple_of(wid * ept, 8), ept)],
                        dst_v)
        for i in range(CH // L):
            ones_v[pl.ds(i * L, L)] = jnp.ones((L,), jnp.float32)

        @pl.when(s == 0)
        def _():
            def zrow(i, carry):
                dbuf[pl.ds(i * L, L)] = jnp.zeros((L,), jnp.float32)
                return carry

            lax.fori_loop(0, N // L, zrow, 0)
            pltpu.sync_copy(dbuf, acc)

        plsc.subcore_barrier()

        def chunk(i, carry):
            pltpu.async_copy(ones_v, acc.at[dst_v.at[pl.ds(i * CH, CH)]],
                             dsem, add=True)
            return carry

        lax.fori_loop(0, cpt, chunk, 0)

        def drain(i, carry):
            pltpu.make_async_copy(ones_v, acc.at[dst_v.at[pl.ds(i * CH, CH)]],
                                  dsem).wait()
            return carry

        lax.fori_loop(0, cpt, drain, 0)
        plsc.subcore_barrier()

        @pl.when(s == 0)
        def _():
            pltpu.sync_copy(acc, dbuf)
            pltpu.sync_copy(dbuf,
                            out_hbm.at[pl.ds(pl.multiple_of(c * N, 8), N)])

    return deg_kernel


def _make_agg(N, E, W, nbuf=NBUF):
    ept = E // NW
    cpt = ept // CH
    base = (N // NS) // 8 * 8
    last = N - base * (NS - 1)

    @functools.partial(
        pl.kernel,
        out_type=jax.ShapeDtypeStruct((NC, N, W), jnp.float32),
        mesh=_mesh(),
        compiler_params=_SC_PARAMS,
        scratch_types=[
            pltpu.VMEM((ept,), jnp.int32),
            pltpu.VMEM((ept,), jnp.int32),
            pltpu.VMEM((nbuf, CH, W), jnp.float32),
            pltpu.VMEM((last, W), jnp.float32),
            pltpu.VMEM_SHARED((N, W), jnp.float32),
            pltpu.SemaphoreType.DMA((nbuf,)),
            pltpu.SemaphoreType.DMA((nbuf,)),
        ],
    )
    def agg_kernel(hp_hbm, edge_hbm, out_hbm,
                   src_v, dst_v, gbuf, bounce, acc, gsem, ssem):
        c = lax.axis_index("c")
        s = lax.axis_index("s")
        wid = c * NS + s
        eoff = pl.multiple_of(wid * ept, 8)
        pltpu.sync_copy(edge_hbm.at[0, pl.ds(eoff, ept)], src_v)

        for b in range(nbuf):
            pltpu.async_copy(hp_hbm.at[src_v.at[pl.ds(b * CH, CH)]],
                             gbuf.at[b], gsem.at[b])

        pltpu.sync_copy(edge_hbm.at[1, pl.ds(eoff, ept)], dst_v)

        def zrow(i, carry):
            for k in range(W // L):
                bounce[i, pl.ds(k * L, L)] = jnp.zeros((L,), jnp.float32)
            return carry

        lax.fori_loop(0, last, zrow, 0)

        @pl.when(s < NS - 1)
        def _():
            off = pl.multiple_of(s * base, 8)
            pltpu.sync_copy(bounce.at[pl.ds(0, base)],
                            acc.at[pl.ds(off, base)])

        @pl.when(s == NS - 1)
        def _():
            pltpu.sync_copy(bounce, acc.at[pl.ds(base * (NS - 1), last)])

        plsc.subcore_barrier()

        def round_(j, carry):
            for b in range(nbuf):
                i = j * nbuf + b
                ic = pl.ds(i * CH, CH)
                nc_ = pl.ds((i + nbuf) * CH, CH)
                pltpu.make_async_copy(hp_hbm.at[src_v.at[ic]], gbuf.at[b],
                                      gsem.at[b]).wait()
                pltpu.async_copy(gbuf.at[b], acc.at[dst_v.at[ic]], ssem.at[b],
                                 add=True)
                pltpu.make_async_copy(gbuf.at[b], acc.at[dst_v.at[ic]],
                                      ssem.at[b]).wait()
                pltpu.async_copy(hp_hbm.at[src_v.at[nc_]], gbuf.at[b],
                                 gsem.at[b])
            return carry

        lax.fori_loop(0, cpt // nbuf - 1, round_, 0)
        for b in range(nbuf):
            ic = pl.ds((cpt - nbuf + b) * CH, CH)
            pltpu.make_async_copy(hp_hbm.at[src_v.at[ic]], gbuf.at[b],
                                  gsem.at[b]).wait()
            pltpu.sync_copy(gbuf.at[b], acc.at[dst_v.at[ic]], add=True)
        plsc.subcore_barrier()

        @pl.when(s < NS - 1)
        def _():
            off = pl.multiple_of(s * base, 8)
            pltpu.sync_copy(acc.at[pl.ds(off, base)],
                            bounce.at[pl.ds(0, base)])
            pltpu.sync_copy(bounce.at[pl.ds(0, base)],
                            out_hbm.at[c, pl.ds(off, base)])

        @pl.when(s == NS - 1)
        def _():
            pltpu.sync_copy(acc.at[pl.ds(base * (NS - 1), last)], bounce)
            pltpu.sync_copy(bounce, out_hbm.at[c, pl.ds(base * (NS - 1), last)])

    return agg_kernel


def _make_score(N, PE, C):
    ept = (2 * PE) // NW
    nb = ept // L
    npass = 2
    hc = C // npass
    base = (N // NS) // 8 * 8
    last = N - base * (NS - 1)

    @functools.partial(
        pl.kernel,
        out_type=jax.ShapeDtypeStruct((2 * PE,), jnp.float32),
        mesh=_mesh(),
        compiler_params=_SC_PARAMS,
        scratch_types=[
            pltpu.VMEM((ept,), jnp.int32),
            pltpu.VMEM((ept,), jnp.int32),
            pltpu.VMEM((ept,), jnp.float32),
            pltpu.VMEM_SHARED((C, N), jnp.float32),
        ],
    )
    def score_kernel(acc2_hbm, hp2_hbm, dinv_hbm, b2_hbm,
                     pos_hbm, neg_hbm,
                     out_hbm, src_v, dst_v, acc_v, h2t):
        c = lax.axis_index("c")
        s = lax.axis_index("s")
        wid = c * NS + s

        @pl.when(wid < NS)
        def _():
            off = pl.multiple_of(wid * ept, 8)
            pltpu.sync_copy(pos_hbm.at[0, pl.ds(off, ept)], src_v)
            pltpu.sync_copy(pos_hbm.at[1, pl.ds(off, ept)], dst_v)

        @pl.when(wid >= NS)
        def _():
            off = pl.multiple_of((wid - NS) * ept, 8)
            pltpu.sync_copy(neg_hbm.at[0, pl.ds(off, ept)], src_v)
            pltpu.sync_copy(neg_hbm.at[1, pl.ds(off, ept)], dst_v)

        def build(a0_v, a1_v, hp_v, dv_v, b2_v, lt_v):
            pltpu.sync_copy(b2_hbm, b2_v)

            @pl.when(s < NS - 1)
            def _():
                off = pl.multiple_of(s * base, 8)
                pltpu.sync_copy(acc2_hbm.at[0, pl.ds(off, base)],
                                a0_v.at[pl.ds(0, base)])
                pltpu.sync_copy(acc2_hbm.at[1, pl.ds(off, base)],
                                a1_v.at[pl.ds(0, base)])
                pltpu.sync_copy(hp2_hbm.at[pl.ds(off, base)],
                                hp_v.at[pl.ds(0, base)])
                pltpu.sync_copy(dinv_hbm.at[pl.ds(off, base)],
                                dv_v.at[pl.ds(0, base)])

            @pl.when(s == NS - 1)
            def _():
                off = base * (NS - 1)
                pltpu.sync_copy(acc2_hbm.at[0, pl.ds(off, last)], a0_v)
                pltpu.sync_copy(acc2_hbm.at[1, pl.ds(off, last)], a1_v)
                pltpu.sync_copy(hp2_hbm.at[pl.ds(off, last)], hp_v)
                pltpu.sync_copy(dinv_hbm.at[pl.ds(off, last)], dv_v)

            ngrp = lax.select(s == NS - 1, last // L, base // L)
            b2row = b2_v[...]
            fidx = lax.iota(jnp.int32, L)
            zil = jnp.zeros((L,), jnp.int32)

            def node16(g, carry):
                dvec = dv_v[pl.ds(g * L, L)]
                for k in range(L):
                    j = g * L + k
                    row = (a0_v[j] + a1_v[j] + hp_v[j]) * dvec[k] + b2row
                    plsc.store_scatter(lt_v, [fidx, zil + j], row)
                return carry

            lax.fori_loop(0, ngrp, node16, 0)

            @pl.when(s < NS - 1)
            def _():
                off = pl.multiple_of(s * base, 8)
                pltpu.sync_copy(lt_v.at[:, pl.ds(0, base)],
                                h2t.at[:, pl.ds(off, base)])

            @pl.when(s == NS - 1)
            def _():
                pltpu.sync_copy(lt_v, h2t.at[:, pl.ds(base * (NS - 1), last)])

        pl.run_scoped(build,
                      pltpu.VMEM((last, C), jnp.float32),
                      pltpu.VMEM((last, C), jnp.float32),
                      pltpu.VMEM((last, C), jnp.float32),
                      pltpu.VMEM((last,), jnp.float32),
                      pltpu.VMEM((C,), jnp.float32),
                      pltpu.VMEM((C, last), jnp.float32))

        plsc.subcore_barrier()

        def gather_phase(half_v):
            for half in range(npass):
                pltpu.sync_copy(h2t.at[pl.ds(half * hc, hc)], half_v)

                def one_block(b):
                    si = src_v[pl.ds(b * L, L)]
                    di = dst_v[pl.ds(b * L, L)]
                    ss = jnp.zeros((L,), jnp.float32)
                    for f in range(hc):
                        vi = plsc.load_gather(half_v.at[f], [di])
                        vj = plsc.load_gather(half_v.at[f], [si])
                        ss = ss + vi * vj
                    if half == 0:
                        acc_v[pl.ds(b * L, L)] = ss
                    else:
                        acc_v[pl.ds(b * L, L)] = acc_v[pl.ds(b * L, L)] + ss

                def block(b2_):
                    for k in range(2):
                        one_block(b2_ * 2 + k)

                plsc.parallel_loop(0, nb // 2, 1, unroll=2)(block)
                for b in range(nb // 2 * 2, nb):
                    one_block(b)

        pl.run_scoped(gather_phase, pltpu.VMEM((hc, N), jnp.float32))

        pltpu.sync_copy(acc_v, out_hbm.at[pl.ds(pl.multiple_of(wid * ept, 8),
                                                ept)])

    return score_kernel


def _mma_body(x_ref, w_ref, h_ref):
    h_ref[...] = jnp.dot(x_ref[...], w_ref[...],
                         preferred_element_type=jnp.float32)


def _mmb_body(h_ref, degp_ref, hp_ref, dinv2_ref, dinv1_ref):
    deg = degp_ref[0, :] + degp_ref[1, :] + 1.0
    dinv = lax.rsqrt(deg)
    hp_ref[...] = h_ref[...] * dinv[:, None]
    dinv2_ref[...] = dinv[:, None]
    dinv1_ref[...] = dinv


def _mm2_body(acc_ref, hp_ref, dinv_ref, b1_ref, w2_ref, hp2_ref):
    dinv = dinv_ref[...]
    h1 = (acc_ref[0] + acc_ref[1] + hp_ref[...]) * dinv + b1_ref[...][None, :]
    h1 = jnp.maximum(h1, 0.0)
    h2 = jnp.dot(h1, w2_ref[...], preferred_element_type=jnp.float32)
    hp2_ref[...] = h2 * dinv


def kernel(x, masked_nodes, pos_edge_index, neg_edge_index, edge_index,
           W1, b1, W2, b2):
    N, F = x.shape
    H = W1.shape[1]
    C = W2.shape[1]
    E = edge_index.shape[1]
    PE = pos_edge_index.shape[1]
    assert E % (NW * CH) == 0 and N % NS == 0 and (2 * PE) % (NW * L) == 0
    assert (E // (NW * CH)) % NBUF == 0 and C == L

    h1raw = pl.pallas_call(
        _mma_body,
        out_shape=jax.ShapeDtypeStruct((N, H), jnp.float32),
    )(x, W1)

    deg_p = _make_deg(N, E)(edge_index)

    hp1, dinv2, dinv1 = pl.pallas_call(
        _mmb_body,
        out_shape=[jax.ShapeDtypeStruct((N, H), jnp.float32),
                   jax.ShapeDtypeStruct((N, 1), jnp.float32),
                   jax.ShapeDtypeStruct((N,), jnp.float32)],
    )(h1raw, deg_p.reshape(NC, N))

    acc1 = _make_agg(N, E, H)(hp1, edge_index)

    hp2 = pl.pallas_call(
        _mm2_body,
        out_shape=jax.ShapeDtypeStruct((N, C), jnp.float32),
    )(acc1, hp1, dinv2, b1, W2)

    acc2 = _make_agg(N, E, C)(hp2, edge_index)

    score = _make_score(N, PE, C)(
        acc2, hp2, dinv1, b2, pos_edge_index, neg_edge_index)
    return (score, jnp.zeros((2, 2 * PE), jnp.float32))
